# baseline (speedup 1.0000x reference)
import functools

import jax
import jax.numpy as jnp
from jax import lax
from jax.experimental import pallas as pl
from jax.experimental.pallas import tpu as pltpu
from jax.experimental.pallas import tpu_sc as plsc

VOCAB = 1000000
EMBED_DIM = 32
BATCH = 4096
SEQ = 200

NC = 2
NS = 16
NW = NC * NS

NB = BATCH // 128
N_GRP = SEQ * NB
PER_W = N_GRP // NW
NBUF = 4
N_REV = PER_W // NBUF


def _body(xg_hbm, table_hbm, out_hbm, idx_v, raws, trs, gsems, wsems, isem):
    wid = lax.axis_index("s") * NC + lax.axis_index("c")
    g0 = wid * PER_W

    pltpu.async_copy(xg_hbm.at[pl.ds(g0, PER_W)], idx_v, isem).wait()

    def fire_gather(j, b):
        pltpu.async_copy(table_hbm.at[idx_v.at[j]], raws[b], gsems[b])

    def wait_gather(b):
        pltpu.make_async_copy(table_hbm.at[idx_v.at[0]], raws[b],
                              gsems[b]).wait()

    def fire_write(j, b):
        g = g0 + j
        s = g // NB
        tb = g % NB
        pltpu.async_copy(trs[b], out_hbm.at[s, :, tb], wsems[b])

    def wait_write(b):
        pltpu.make_async_copy(trs[b], out_hbm.at[0, :, 0], wsems[b]).wait()

    iotas = [jnp.arange(c0 * 16, c0 * 16 + 16, dtype=jnp.int32)
             for c0 in range(8)]

    def transpose(b):
        raw, tr = raws[b], trs[b]

        def te_body(te, carry):
            for ep in range(8):
                e = te * 8 + ep
                ev = jnp.full((16,), 0, jnp.int32) + e
                for c0 in range(8):
                    v = plsc.load_gather(raw, [iotas[c0], ev])
                    tr[te, ep, pl.ds(c0 * 16, 16)] = v
            return carry

        lax.fori_loop(0, 4, te_body, 0)

    for b in range(NBUF):
        fire_gather(b, b)

    def rev(g, carry):
        for b in range(NBUF):
            j = g * NBUF + b
            wait_gather(b)
            transpose(b)
            fire_write(j, b)
        for b in range(NBUF):
            j = g * NBUF + b
            wait_write(b)

            @pl.when(g < N_REV - 1)
            def _():
                fire_gather(j + NBUF, b)

        return carry

    lax.fori_loop(0, N_REV, rev, 0)


@functools.partial(jax.jit, static_argnames=())
def kernel(x, table):
    xg = x.T.reshape(N_GRP, 128)
    mesh = plsc.VectorSubcoreMesh(core_axis_name="c", subcore_axis_name="s")
    out5 = pl.kernel(
        _body,
        out_type=jax.ShapeDtypeStruct((SEQ, 4, NB, 8, 128), jnp.float32),
        mesh=mesh,
        compiler_params=pltpu.CompilerParams(use_tc_tiling_on_sc=False,
                                             needs_layout_passes=False),
        scratch_types=[
            pltpu.VMEM((PER_W, 128), jnp.int32),
            [pltpu.VMEM((128, EMBED_DIM), jnp.float32) for _ in range(NBUF)],
            [pltpu.VMEM((4, 8, 128), jnp.float32) for _ in range(NBUF)],
            [pltpu.SemaphoreType.DMA for _ in range(NBUF)],
            [pltpu.SemaphoreType.DMA for _ in range(NBUF)],
            pltpu.SemaphoreType.DMA,
        ],
    )(xg, table)
    return out5.transpose(2, 4, 0, 1, 3).reshape(BATCH, SEQ, EMBED_DIM)


# speedup vs baseline: 1.0024x; 1.0024x over previous
import functools

import jax
import jax.numpy as jnp
from jax import lax
from jax.experimental import pallas as pl
from jax.experimental.pallas import tpu as pltpu
from jax.experimental.pallas import tpu_sc as plsc

VOCAB = 1000000
EMBED_DIM = 32
BATCH = 4096
SEQ = 200

NC = 2
NS = 16
NW = NC * NS

NB = BATCH // 128
N_GRP = SEQ * NB
PER_W = N_GRP // NW
NBUF = 4
N_REV = PER_W // NBUF


def _body(xg_hbm, table_hbm, out_hbm, idx_v, raws, trs, gsems, wsems, isem):
    wid = lax.axis_index("s") * NC + lax.axis_index("c")
    g0 = wid * PER_W

    pltpu.async_copy(xg_hbm.at[pl.ds(g0, PER_W)], idx_v, isem).wait()

    def fire_gather(j, b):
        pltpu.async_copy(table_hbm.at[idx_v.at[j]], raws[b], gsems[b])

    def wait_gather(b):
        pltpu.make_async_copy(table_hbm.at[idx_v.at[0]], raws[b],
                              gsems[b]).wait()

    def fire_write(j, b):
        g = g0 + j
        s = g // NB
        tb = g % NB
        pltpu.async_copy(trs[b], out_hbm.at[s, :, tb], wsems[b])

    def wait_write(b):
        pltpu.make_async_copy(trs[b], out_hbm.at[0, :, 0], wsems[b]).wait()

    iotac = [jnp.arange(c0 * 16, c0 * 16 + 16, dtype=jnp.int32)
             for c0 in range(8)]

    def transpose(b):
        raw, tr = raws[b], trs[b]

        def te_body(te, carry):
            for ep in range(8):
                e16 = jnp.broadcast_to(te * 8 + ep, (16,)).astype(jnp.int32)
                for c0 in range(8):
                    v = plsc.load_gather(raw, [iotac[c0], e16])
                    tr[te, ep, pl.ds(c0 * 16, 16)] = v
            return carry

        lax.fori_loop(0, 4, te_body, 0)

    for b in range(NBUF):
        fire_gather(b, b)

    def rev(g, carry):
        for b in range(NBUF):
            j = g * NBUF + b
            wait_gather(b)
            transpose(b)
            fire_write(j, b)
        for b in range(NBUF):
            j = g * NBUF + b
            wait_write(b)

            @pl.when(g < N_REV - 1)
            def _():
                fire_gather(j + NBUF, b)

        return carry

    lax.fori_loop(0, N_REV, rev, 0)


@functools.partial(jax.jit, static_argnames=())
def kernel(x, table):
    xg = x.T.reshape(N_GRP, 128)
    mesh = plsc.VectorSubcoreMesh(core_axis_name="c", subcore_axis_name="s")
    out5 = pl.kernel(
        _body,
        out_type=jax.ShapeDtypeStruct((SEQ, 4, NB, 8, 128), jnp.float32),
        mesh=mesh,
        compiler_params=pltpu.CompilerParams(use_tc_tiling_on_sc=False,
                                             needs_layout_passes=False),
        scratch_types=[
            pltpu.VMEM((PER_W, 128), jnp.int32),
            [pltpu.VMEM((128, EMBED_DIM), jnp.float32) for _ in range(NBUF)],
            [pltpu.VMEM((4, 8, 128), jnp.float32) for _ in range(NBUF)],
            [pltpu.SemaphoreType.DMA for _ in range(NBUF)],
            [pltpu.SemaphoreType.DMA for _ in range(NBUF)],
            pltpu.SemaphoreType.DMA,
        ],
    )(xg, table)
    return out5.transpose(2, 4, 0, 1, 3).reshape(BATCH, SEQ, EMBED_DIM)


# speedup vs baseline: 1.6089x; 1.6051x over previous
import functools

import jax
import jax.numpy as jnp
from jax import lax
from jax.experimental import pallas as pl
from jax.experimental.pallas import tpu as pltpu
from jax.experimental.pallas import tpu_sc as plsc

VOCAB = 1000000
EMBED_DIM = 32
BATCH = 4096
SEQ = 200

NC = 2
NS = 16
NW = NC * NS

NB = BATCH // 128
N_GRP = SEQ * NB
PER_W = N_GRP // NW
NBUF = 4
N_REV = PER_W // NBUF


def _body(xg_hbm, table_hbm, out_hbm, idx_v, raws, trs, gsems, wsems, isem):
    wid = lax.axis_index("s") * NC + lax.axis_index("c")
    g0 = wid * PER_W

    pltpu.async_copy(xg_hbm.at[pl.ds(g0, PER_W)], idx_v, isem).wait()

    def fire_gather(j, b):
        pltpu.async_copy(table_hbm.at[idx_v.at[j]], raws[b], gsems[b])

    def wait_gather(b):
        pltpu.make_async_copy(table_hbm.at[idx_v.at[0]], raws[b],
                              gsems[b]).wait()

    def fire_write(j, b):
        g = g0 + j
        s = g // NB
        tb = g % NB
        pltpu.async_copy(trs[b], out_hbm.at[s, :, tb], wsems[b])

    def wait_write(b):
        pltpu.make_async_copy(trs[b], out_hbm.at[0, :, 0], wsems[b]).wait()

    iotac = [jnp.arange(c0 * 16, c0 * 16 + 16, dtype=jnp.int32)
             for c0 in range(8)]

    def transpose(b):
        raw, tr = raws[b], trs[b]

        def d_body(d, carry):
            dv = jnp.broadcast_to(d, (16,)).astype(jnp.int32)
            for c0 in range(8):
                ev = (iotac[c0] + dv) & 31
                v = plsc.load_gather(raw, [iotac[c0], ev])
                plsc.store_scatter(
                    tr, [ev >> 3, ev & 7, iotac[c0]], v)
            return carry

        lax.fori_loop(0, EMBED_DIM, d_body, 0)

    for b in range(NBUF):
        fire_gather(b, b)

    def rev(g, carry):
        for b in range(NBUF):
            j = g * NBUF + b
            wait_gather(b)
            transpose(b)
            fire_write(j, b)
        for b in range(NBUF):
            j = g * NBUF + b
            wait_write(b)

            @pl.when(g < N_REV - 1)
            def _():
                fire_gather(j + NBUF, b)

        return carry

    lax.fori_loop(0, N_REV, rev, 0)


@functools.partial(jax.jit, static_argnames=())
def kernel(x, table):
    xg = x.T.reshape(N_GRP, 128)
    mesh = plsc.VectorSubcoreMesh(core_axis_name="c", subcore_axis_name="s")
    out5 = pl.kernel(
        _body,
        out_type=jax.ShapeDtypeStruct((SEQ, 4, NB, 8, 128), jnp.float32),
        mesh=mesh,
        compiler_params=pltpu.CompilerParams(use_tc_tiling_on_sc=False,
                                             needs_layout_passes=False),
        scratch_types=[
            pltpu.VMEM((PER_W, 128), jnp.int32),
            [pltpu.VMEM((128, EMBED_DIM), jnp.float32) for _ in range(NBUF)],
            [pltpu.VMEM((4, 8, 128), jnp.float32) for _ in range(NBUF)],
            [pltpu.SemaphoreType.DMA for _ in range(NBUF)],
            [pltpu.SemaphoreType.DMA for _ in range(NBUF)],
            pltpu.SemaphoreType.DMA,
        ],
    )(xg, table)
    return out5.transpose(2, 4, 0, 1, 3).reshape(BATCH, SEQ, EMBED_DIM)


# speedup vs baseline: 1.6437x; 1.0216x over previous
import functools

import jax
import jax.numpy as jnp
from jax import lax
from jax.experimental import pallas as pl
from jax.experimental.pallas import tpu as pltpu
from jax.experimental.pallas import tpu_sc as plsc

VOCAB = 1000000
EMBED_DIM = 32
BATCH = 4096
SEQ = 200

NC = 2
NS = 16
NW = NC * NS

NB = BATCH // 128
N_GRP = SEQ * NB
PER_W = N_GRP // NW
NBUF = 8
N_REV = PER_W // NBUF


def _body(xg_hbm, table_hbm, out_hbm, idx_v, raws, trs, gsems, wsems, isem):
    wid = lax.axis_index("s") * NC + lax.axis_index("c")
    g0 = wid * PER_W

    pltpu.async_copy(xg_hbm.at[pl.ds(g0, PER_W)], idx_v, isem).wait()

    def fire_gather(j, b):
        pltpu.async_copy(table_hbm.at[idx_v.at[j]], raws[b], gsems[b])

    def wait_gather(b):
        pltpu.make_async_copy(table_hbm.at[idx_v.at[0]], raws[b],
                              gsems[b]).wait()

    def fire_write(j, b):
        g = g0 + j
        s = g // NB
        tb = g % NB
        pltpu.async_copy(trs[b], out_hbm.at[s, :, tb], wsems[b])

    def wait_write(b):
        pltpu.make_async_copy(trs[b], out_hbm.at[0, :, 0], wsems[b]).wait()

    iotac = [jnp.arange(c0 * 16, c0 * 16 + 16, dtype=jnp.int32)
             for c0 in range(8)]

    def transpose(b):
        raw, tr = raws[b], trs[b]

        def d_body(d, carry):
            dv = jnp.broadcast_to(d, (16,)).astype(jnp.int32)
            for c0 in range(8):
                ev = (iotac[c0] + dv) & 31
                v = plsc.load_gather(raw, [iotac[c0], ev])
                plsc.store_scatter(
                    tr, [ev >> 3, ev & 7, iotac[c0]], v)
            return carry

        lax.fori_loop(0, EMBED_DIM, d_body, 0)

    for b in range(NBUF):
        fire_gather(b, b)

    def rev(g, carry):
        for b in range(NBUF):
            j = g * NBUF + b
            wait_gather(b)
            transpose(b)
            fire_write(j, b)
        for b in range(NBUF):
            j = g * NBUF + b
            wait_write(b)

            @pl.when(g < N_REV - 1)
            def _():
                fire_gather(j + NBUF, b)

        return carry

    lax.fori_loop(0, N_REV, rev, 0)


@functools.partial(jax.jit, static_argnames=())
def kernel(x, table):
    xg = x.T.reshape(N_GRP, 128)
    mesh = plsc.VectorSubcoreMesh(core_axis_name="c", subcore_axis_name="s")
    out5 = pl.kernel(
        _body,
        out_type=jax.ShapeDtypeStruct((SEQ, 4, NB, 8, 128), jnp.float32),
        mesh=mesh,
        compiler_params=pltpu.CompilerParams(use_tc_tiling_on_sc=False,
                                             needs_layout_passes=False),
        scratch_types=[
            pltpu.VMEM((PER_W, 128), jnp.int32),
            [pltpu.VMEM((128, EMBED_DIM), jnp.float32) for _ in range(NBUF)],
            [pltpu.VMEM((4, 8, 128), jnp.float32) for _ in range(NBUF)],
            [pltpu.SemaphoreType.DMA for _ in range(NBUF)],
            [pltpu.SemaphoreType.DMA for _ in range(NBUF)],
            pltpu.SemaphoreType.DMA,
        ],
    )(xg, table)
    return out5.transpose(2, 4, 0, 1, 3).reshape(BATCH, SEQ, EMBED_DIM)
